# baseline (speedup 1.0000x reference)
import functools
import jax
import jax.numpy as jnp
import numpy as np
from jax.experimental import pallas as pl

HID = 256


def _mm_kernel(x_ref, w_ref, b_ref, o_ref):
    o_ref[...] = (
        jnp.dot(x_ref[...], w_ref[...], preferred_element_type=jnp.float32)
        + b_ref[...]
    )


def _matmul_bias(x, w, b, bm=2000):
    M, K = x.shape
    F = w.shape[1]
    Mp = ((M + bm - 1) // bm) * bm
    if Mp != M:
        x = jnp.pad(x, ((0, Mp - M), (0, 0)))
    out = pl.pallas_call(
        _mm_kernel,
        grid=(Mp // bm,),
        in_specs=[
            pl.BlockSpec((bm, K), lambda i: (i, 0)),
            pl.BlockSpec((K, F), lambda i: (0, 0)),
            pl.BlockSpec((1, F), lambda i: (0, 0)),
        ],
        out_specs=pl.BlockSpec((bm, F), lambda i: (i, 0)),
        out_shape=jax.ShapeDtypeStruct((Mp, F), jnp.float32),
    )(x, w, b[None, :])
    return out[:M]


def _batchnorm(x, g, b, eps=1e-5):
    mu = jnp.mean(x, axis=0, keepdims=True)
    var = jnp.var(x, axis=0, keepdims=True)
    return g * (x - mu) / jnp.sqrt(var + eps) + b


def _mlp_apply(p, x):
    return jax.nn.silu(_batchnorm(_matmul_bias(x, p['w'], p['b']), p['g'], p['be']))


def _rbf(d, vmin, vmax, bins):
    centers = jnp.linspace(vmin, vmax, bins)
    gamma = 1.0 / ((vmax - vmin) / (bins - 1))
    return jnp.exp(-gamma * (d[:, None] - centers[None, :]) ** 2)


def _eggc_apply(p, src, dst, x, y, n_nodes, sorted_dst=False):
    w4 = jnp.concatenate(
        [p['src_gate_w'], p['dst_gate_w'], p['dst_update_w'], p['src_update_w']],
        axis=1)
    b4 = jnp.concatenate(
        [p['src_gate_b'], p['dst_gate_b'], p['dst_update_b'], p['src_update_b']])
    x4 = _matmul_bias(x, w4, b4)
    e_src = x4[:, 0:HID]
    e_dst = x4[:, HID:2 * HID]
    bh = x4[:, 2 * HID:3 * HID]
    xup = x4[:, 3 * HID:4 * HID]
    yg = _matmul_bias(y, p['edge_gate_w'], p['edge_gate_b'])

    m = e_src[src] + e_dst[dst] + yg
    sigma = jax.nn.sigmoid(m)
    cat = jnp.concatenate([sigma * bh[src], sigma], axis=1)
    ss = jax.ops.segment_sum(
        cat, dst, num_segments=n_nodes, indices_are_sorted=sorted_dst)
    h = ss[:, :HID] / (ss[:, HID:] + 1e-6)
    xn = jax.nn.silu(_batchnorm(xup + h, p['bn_nodes_g'], p['bn_nodes_b']))
    yn = jax.nn.silu(_batchnorm(m, p['bn_edges_g'], p['bn_edges_b']))
    return x + xn, y + yn


def kernel(atom_features, r, angle_h, edge_index, lg_edge_index, params):
    src, dst = edge_index[0], edge_index[1]
    lsrc, ldst = lg_edge_index[0], lg_edge_index[1]
    n_nodes = atom_features.shape[0]
    n_edges = r.shape[0]

    lperm = jnp.argsort(ldst)
    lsrc = lsrc[lperm]
    ldst = ldst[lperm]

    x = _mlp_apply(params['atom_emb'], atom_features)
    bondlength = jnp.linalg.norm(r, axis=1)
    y = _rbf(bondlength, 0.0, 8.0, 80)
    y = _mlp_apply(params['edge_emb2'], _mlp_apply(params['edge_emb1'], y))
    z = _rbf(angle_h, -1.0, 1.0, 40)
    z = _mlp_apply(params['angle_emb2'], _mlp_apply(params['angle_emb1'], z))
    z = z[lperm]
    for lp in params['alignn']:
        x, m = _eggc_apply(lp['node'], src, dst, x, y, n_nodes)
        y, z = _eggc_apply(lp['edge'], lsrc, ldst, m, z, n_edges,
                           sorted_dst=True)
    for lp in params['gcn']:
        x, y = _eggc_apply(lp, src, dst, x, y, n_nodes)
    h = jnp.mean(x, axis=0)
    out = h @ params['fc_w'] + params['fc_b']
    return jnp.squeeze(out)


# speedup vs baseline: 1.0512x; 1.0512x over previous
import functools
import jax
import jax.numpy as jnp
import numpy as np
from jax.experimental import pallas as pl

HID = 256


def _mm_kernel(x_ref, w_ref, b_ref, o_ref):
    o_ref[...] = (
        jnp.dot(x_ref[...], w_ref[...], preferred_element_type=jnp.float32)
        + b_ref[...]
    )


def _matmul_bias(x, w, b, bm=2000):
    M, K = x.shape
    F = w.shape[1]
    Mp = ((M + bm - 1) // bm) * bm
    if Mp != M:
        x = jnp.pad(x, ((0, Mp - M), (0, 0)))
    out = pl.pallas_call(
        _mm_kernel,
        grid=(Mp // bm,),
        in_specs=[
            pl.BlockSpec((bm, K), lambda i: (i, 0)),
            pl.BlockSpec((K, F), lambda i: (0, 0)),
            pl.BlockSpec((1, F), lambda i: (0, 0)),
        ],
        out_specs=pl.BlockSpec((bm, F), lambda i: (i, 0)),
        out_shape=jax.ShapeDtypeStruct((Mp, F), jnp.float32),
    )(x, w, b[None, :])
    return out[:M]


def _batchnorm(x, g, b, eps=1e-5):
    mu = jnp.mean(x, axis=0, keepdims=True)
    var = jnp.var(x, axis=0, keepdims=True)
    return g * (x - mu) / jnp.sqrt(var + eps) + b


def _mlp_apply(p, x):
    return jax.nn.silu(_batchnorm(_matmul_bias(x, p['w'], p['b']), p['g'], p['be']))


def _rbf(d, vmin, vmax, bins):
    centers = jnp.linspace(vmin, vmax, bins)
    gamma = 1.0 / ((vmax - vmin) / (bins - 1))
    return jnp.exp(-gamma * (d[:, None] - centers[None, :]) ** 2)


def _eggc_apply(p, src, dst, x, y, n_nodes, sorted_dst=False):
    w4 = jnp.concatenate(
        [p['src_gate_w'], p['dst_gate_w'], p['dst_update_w'], p['src_update_w']],
        axis=1)
    b4 = jnp.concatenate(
        [p['src_gate_b'], p['dst_gate_b'], p['dst_update_b'], p['src_update_b']])
    x4 = _matmul_bias(x, w4, b4)
    e_src = x4[:, 0:HID]
    e_dst = x4[:, HID:2 * HID]
    bh = x4[:, 2 * HID:3 * HID]
    xup = x4[:, 3 * HID:4 * HID]
    yg = _matmul_bias(y, p['edge_gate_w'], p['edge_gate_b'])

    m = e_src[src] + e_dst[dst] + yg
    sigma = jax.nn.sigmoid(m)
    sum_sigma_h = jax.ops.segment_sum(
        sigma * bh[src], dst, num_segments=n_nodes,
        indices_are_sorted=sorted_dst)
    sum_sigma = jax.ops.segment_sum(
        sigma, dst, num_segments=n_nodes, indices_are_sorted=sorted_dst)
    h = sum_sigma_h / (sum_sigma + 1e-6)
    xn = jax.nn.silu(_batchnorm(xup + h, p['bn_nodes_g'], p['bn_nodes_b']))
    yn = jax.nn.silu(_batchnorm(m, p['bn_edges_g'], p['bn_edges_b']))
    return x + xn, y + yn


def kernel(atom_features, r, angle_h, edge_index, lg_edge_index, params):
    src, dst = edge_index[0], edge_index[1]
    lsrc, ldst = lg_edge_index[0], lg_edge_index[1]
    n_nodes = atom_features.shape[0]
    n_edges = r.shape[0]

    x = _mlp_apply(params['atom_emb'], atom_features)
    bondlength = jnp.linalg.norm(r, axis=1)
    y = _rbf(bondlength, 0.0, 8.0, 80)
    y = _mlp_apply(params['edge_emb2'], _mlp_apply(params['edge_emb1'], y))
    z = _rbf(angle_h, -1.0, 1.0, 40)
    z = _mlp_apply(params['angle_emb2'], _mlp_apply(params['angle_emb1'], z))
    for lp in params['alignn']:
        x, m = _eggc_apply(lp['node'], src, dst, x, y, n_nodes)
        y, z = _eggc_apply(lp['edge'], lsrc, ldst, m, z, n_edges)
    for lp in params['gcn']:
        x, y = _eggc_apply(lp, src, dst, x, y, n_nodes)
    h = jnp.mean(x, axis=0)
    out = h @ params['fc_w'] + params['fc_b']
    return jnp.squeeze(out)


# speedup vs baseline: 1.0664x; 1.0145x over previous
import functools
import jax
import jax.numpy as jnp
import numpy as np
from jax.experimental import pallas as pl

HID = 256


def _mm_kernel(x_ref, w_ref, b_ref, o_ref):
    o_ref[...] = (
        jnp.dot(x_ref[...], w_ref[...], preferred_element_type=jnp.float32)
        + b_ref[...]
    )


def _matmul_bias(x, w, b, bm=2000):
    M, K = x.shape
    F = w.shape[1]
    Mp = ((M + bm - 1) // bm) * bm
    if Mp != M:
        x = jnp.pad(x, ((0, Mp - M), (0, 0)))
    out = pl.pallas_call(
        _mm_kernel,
        grid=(Mp // bm,),
        in_specs=[
            pl.BlockSpec((bm, K), lambda i: (i, 0)),
            pl.BlockSpec((K, F), lambda i: (0, 0)),
            pl.BlockSpec((1, F), lambda i: (0, 0)),
        ],
        out_specs=pl.BlockSpec((bm, F), lambda i: (i, 0)),
        out_shape=jax.ShapeDtypeStruct((Mp, F), jnp.float32),
    )(x, w, b[None, :])
    return out[:M]


def _batchnorm(x, g, b, eps=1e-5):
    mu = jnp.mean(x, axis=0, keepdims=True)
    var = jnp.var(x, axis=0, keepdims=True)
    return g * (x - mu) / jnp.sqrt(var + eps) + b


def _mlp_apply(p, x):
    return jax.nn.silu(_batchnorm(_matmul_bias(x, p['w'], p['b']), p['g'], p['be']))


def _rbf(d, vmin, vmax, bins):
    centers = jnp.linspace(vmin, vmax, bins)
    gamma = 1.0 / ((vmax - vmin) / (bins - 1))
    return jnp.exp(-gamma * (d[:, None] - centers[None, :]) ** 2)


def _eggc_apply(p, src, dst, x, y, n_nodes, sorted_dst=False):
    w4 = jnp.concatenate(
        [p['src_gate_w'], p['dst_gate_w'], p['dst_update_w'], p['src_update_w']],
        axis=1)
    b4 = jnp.concatenate(
        [p['src_gate_b'], p['dst_gate_b'], p['dst_update_b'], p['src_update_b']])
    x4 = _matmul_bias(x, w4, b4)
    e_src = x4[:, 0:HID]
    e_dst = x4[:, HID:2 * HID]
    bh = x4[:, 2 * HID:3 * HID]
    xup = x4[:, 3 * HID:4 * HID]
    yg = _matmul_bias(y, p['edge_gate_w'], p['edge_gate_b'])

    m = e_src[src] + e_dst[dst] + yg
    sigma = jax.nn.sigmoid(m)
    sum_sigma_h = jax.ops.segment_sum(
        (sigma * bh[src]).astype(jnp.bfloat16), dst, num_segments=n_nodes,
        indices_are_sorted=sorted_dst).astype(jnp.float32)
    sum_sigma = jax.ops.segment_sum(
        sigma.astype(jnp.bfloat16), dst, num_segments=n_nodes,
        indices_are_sorted=sorted_dst).astype(jnp.float32)
    h = sum_sigma_h / (sum_sigma + 1e-6)
    xn = jax.nn.silu(_batchnorm(xup + h, p['bn_nodes_g'], p['bn_nodes_b']))
    yn = jax.nn.silu(_batchnorm(m, p['bn_edges_g'], p['bn_edges_b']))
    return x + xn, y + yn


def kernel(atom_features, r, angle_h, edge_index, lg_edge_index, params):
    src, dst = edge_index[0], edge_index[1]
    lsrc, ldst = lg_edge_index[0], lg_edge_index[1]
    n_nodes = atom_features.shape[0]
    n_edges = r.shape[0]

    x = _mlp_apply(params['atom_emb'], atom_features)
    bondlength = jnp.linalg.norm(r, axis=1)
    y = _rbf(bondlength, 0.0, 8.0, 80)
    y = _mlp_apply(params['edge_emb2'], _mlp_apply(params['edge_emb1'], y))
    z = _rbf(angle_h, -1.0, 1.0, 40)
    z = _mlp_apply(params['angle_emb2'], _mlp_apply(params['angle_emb1'], z))
    for lp in params['alignn']:
        x, m = _eggc_apply(lp['node'], src, dst, x, y, n_nodes)
        y, z = _eggc_apply(lp['edge'], lsrc, ldst, m, z, n_edges)
    for lp in params['gcn']:
        x, y = _eggc_apply(lp, src, dst, x, y, n_nodes)
    h = jnp.mean(x, axis=0)
    out = h @ params['fc_w'] + params['fc_b']
    return jnp.squeeze(out)
